# baseline (speedup 1.0000x reference)
import functools

import jax
import jax.numpy as jnp
from jax.experimental import pallas as pl
from jax.experimental.pallas import tpu as pltpu

_R = 1000


def _tc1_body(pos_ref, s2_ref, h_ref, We1_ref, be1_ref, We2_ref, be2_ref,
              Wm_ref, bm_ref, A_ref, B_ref, z_ref):
    D = h_ref.shape[1]
    x1 = pos_ref[...]
    x2 = s2_ref[...]
    z1 = jnp.maximum(x1 * We1_ref[0:1, :] + x2 * We1_ref[1:2, :] + be1_ref[...], 0.0)
    z = jnp.dot(z1, We2_ref[...], preferred_element_type=jnp.float32) + be2_ref[...]
    h = h_ref[...]
    A_ref[...] = (jnp.dot(z, Wm_ref[0:D, :], preferred_element_type=jnp.float32)
                  + jnp.dot(h, Wm_ref[D:2 * D, :], preferred_element_type=jnp.float32))
    B_ref[...] = (jnp.dot(z, Wm_ref[2 * D:3 * D, :], preferred_element_type=jnp.float32)
                  + jnp.dot(h, Wm_ref[3 * D:4 * D, :], preferred_element_type=jnp.float32)
                  + bm_ref[...])
    z_ref[...] = z


def _tc1(pos_c, s2_c, h, We1, be1r, We2, be2r, Wm, bmr):
    N, D = h.shape
    grid = N // _R
    row = lambda i: (i, 0)
    full = lambda i: (0, 0)
    return pl.pallas_call(
        _tc1_body,
        grid=(grid,),
        in_specs=[
            pl.BlockSpec((_R, 1), row),
            pl.BlockSpec((_R, 1), row),
            pl.BlockSpec((_R, D), row),
            pl.BlockSpec((2, D), full),
            pl.BlockSpec((1, D), full),
            pl.BlockSpec((D, D), full),
            pl.BlockSpec((1, D), full),
            pl.BlockSpec((4 * D, D), full),
            pl.BlockSpec((1, D), full),
        ],
        out_specs=[pl.BlockSpec((_R, D), row)] * 3,
        out_shape=[jax.ShapeDtypeStruct((N, D), jnp.float32)] * 3,
    )(pos_c, s2_c, h, We1, be1r, We2, be2r, Wm, bmr)


def _tc2_body(z_ref, h_ref, agg_ref, Wu_ref, Wa_ref, bu_ref, Wd2_ref, bd2_ref,
              hn_ref, uv_ref):
    D = h_ref.shape[1]
    z = z_ref[...]
    h = h_ref[...]
    agg = agg_ref[...]
    hn = jnp.maximum(
        jnp.dot(z, Wu_ref[0:D, :], preferred_element_type=jnp.float32)
        + jnp.dot(h, Wu_ref[D:2 * D, :], preferred_element_type=jnp.float32)
        + jnp.dot(agg, Wa_ref[...], preferred_element_type=jnp.float32)
        + bu_ref[...], 0.0)
    hn_ref[...] = hn
    uv_ref[...] = jnp.dot(hn, Wd2_ref[...], preferred_element_type=jnp.float32) + bd2_ref[...]


def _tc2(z, h, agg, Wu, Wa, bur, Wd2, bd2):
    N, D = h.shape
    grid = N // _R
    row = lambda i: (i, 0)
    full = lambda i: (0, 0)
    return pl.pallas_call(
        _tc2_body,
        grid=(grid,),
        in_specs=[
            pl.BlockSpec((_R, D), row),
            pl.BlockSpec((_R, D), row),
            pl.BlockSpec((_R, D), row),
            pl.BlockSpec((2 * D, D), full),
            pl.BlockSpec((D, D), full),
            pl.BlockSpec((1, D), full),
            pl.BlockSpec((D, 2), full),
            pl.BlockSpec((1, 2), full),
        ],
        out_specs=[pl.BlockSpec((_R, D), row), pl.BlockSpec((_R, 2), row)],
        out_shape=[jax.ShapeDtypeStruct((N, D), jnp.float32),
                   jax.ShapeDtypeStruct((N, 2), jnp.float32)],
    )(z, h, agg, Wu, Wa, bur, Wd2, bd2)


def _loss_body(t0_ref, t1_ref, ef_ref, h0_ref, h1_ref, reach_ref, rt_ref,
               nmask_ref, par_ref, pi_ref, lx_ref, lh_ref, lr_ref, lp_ref):
    E = t0_ref.shape[0] * t0_ref.shape[1]
    Nn = 10000.0

    def bce_sum(t, tgt):
        p = 1.0 / (1.0 + jnp.exp(-t))
        p = jnp.clip(p, 1e-7, 1.0 - 1e-7)
        return -jnp.sum(tgt * jnp.log(p) + (1.0 - tgt) * jnp.log(1.0 - p))

    t0 = t0_ref[...]
    t1 = t1_ref[...]
    lx_ref[...] = (bce_sum(t1, ef_ref[...]) / E).reshape(1, 1)
    lh_ref[...] = ((bce_sum(t0, h0_ref[...]) + bce_sum(t1, h1_ref[...])) / E).reshape(1, 1)
    pr = jnp.clip(reach_ref[...], 1e-7, 1.0 - 1e-7)
    rt = rt_ref[...]
    m = nmask_ref[...]
    lr_ref[...] = (-jnp.sum(m * (rt * jnp.log(pr) + (1.0 - rt) * jnp.log(1.0 - pr))) / Nn).reshape(1, 1)
    lp_ref[...] = (1.0 - jnp.sum((par_ref[...] == pi_ref[...]).astype(jnp.float32)) / Nn).reshape(1, 1)


def _tc3(t0, t1, ef, h0, h1, reach_p, rt_p, nmask, par_p, pi_p):
    scal = jax.ShapeDtypeStruct((1, 1), jnp.float32)
    outs = pl.pallas_call(
        _loss_body,
        out_shape=[scal] * 4,
    )(t0, t1, ef, h0, h1, reach_p, rt_p, nmask, par_p, pi_p)
    return tuple(o.reshape(()) for o in outs)


def kernel(pos, s, edge_index, edges, edges_h, reach_h, pi,
           We1, be1, We2, be2, Wm, bm, Wu, Wa, bu, Wd, bd):
    N = pos.shape[0]
    E = edge_index.shape[1]
    D = We1.shape[1]
    T = edges_h.shape[0]
    max_iter = T - 1
    src = edge_index[0]
    dst = edge_index[1]

    pos_c = pos.reshape(N, 1)
    be1r = be1.reshape(1, D)
    be2r = be2.reshape(1, D)
    bmr = bm.reshape(1, D)
    bur = bu.reshape(1, D)
    Wd2 = jnp.concatenate([Wd[:D], Wd[D:]], axis=1)
    bd2 = jnp.stack([jnp.zeros((), jnp.float32), bd[0]]).reshape(1, 2)
    THR = jnp.float32(jnp.log(0.4 / 0.6))

    h = jnp.zeros((N, D), jnp.float32)
    s2 = s.reshape(N, 1)
    ts = []
    reach = None
    for i in range(max_iter):
        A, B, z = _tc1(pos_c, s2, h, We1, be1r, We2, be2r, Wm, bmr)
        msg = jnp.maximum(A[src] + B[dst], 0.0)
        agg = jax.ops.segment_sum(msg, dst, num_segments=N)
        h, uv = _tc2(z, h, agg, Wu, Wa, bur, Wd2, bd2)
        t = uv[src, 0] + uv[dst, 1]
        ts.append(t)
        ind = (t >= THR).astype(jnp.float32)
        cnt = (jax.ops.segment_sum(ind, src, num_segments=N)
               + jax.ops.segment_sum(ind, dst, num_segments=N))
        reach = (cnt > 0.0).astype(jnp.float32)
        s2 = reach.reshape(N, 1)

    alpha = jax.nn.sigmoid(ts[-1])
    seg_max = jax.ops.segment_max(alpha, dst, num_segments=N)
    match = jnp.isfinite(seg_max)[dst] & (alpha == seg_max[dst])
    eidx = jnp.where(match, jnp.arange(E, dtype=jnp.int32), jnp.int32(E))
    first = jax.ops.segment_min(eidx, dst, num_segments=N)
    first = jnp.minimum(first, jnp.int32(E))
    parents = jnp.where(first < E, src[jnp.clip(first, 0, E - 1)],
                        jnp.arange(N, dtype=src.dtype))

    Eb = E // D
    t0r = ts[0].reshape(Eb, D)
    t1r = ts[1].reshape(Eb, D)
    ef = edges.astype(jnp.float32).reshape(Eb, D)
    h0 = edges_h[1].astype(jnp.float32).reshape(Eb, D)
    h1 = edges_h[2].astype(jnp.float32).reshape(Eb, D)
    Np = 10240
    pad = Np - N

    def padf(x, val):
        return jnp.concatenate([x, jnp.full((pad,), val, x.dtype)]).reshape(Np // D, D)

    reach_p = padf(reach, 0.5)
    rt_p = padf(reach_h[-1].astype(jnp.float32), 0.5)
    nmask = padf(jnp.ones((N,), jnp.float32), 0.0)
    par_p = padf(parents.astype(jnp.int32), -2)
    pi_p = padf(pi.astype(jnp.int32), -1)
    return _tc3(t0r, t1r, ef, h0, h1, reach_p, rt_p, nmask, par_p, pi_p)


# speedup vs baseline: 1.1047x; 1.1047x over previous
import functools

import jax
import jax.numpy as jnp
from jax import lax
from jax.experimental import pallas as pl
from jax.experimental.pallas import tpu as pltpu
from jax.experimental.pallas import tpu_sc as plsc

_NP = 10240
_R = 1024
_D = 128
_NW = 32
_CHUNK = 128


def _tc1_body(pos_ref, s2_ref, h_ref, We1_ref, be1_ref, We2_ref, be2_ref,
              Wm_ref, bm_ref, A_ref, B_ref, z_ref):
    D = h_ref.shape[1]
    x1 = pos_ref[...]
    x2 = s2_ref[...]
    z1 = jnp.maximum(x1 * We1_ref[0:1, :] + x2 * We1_ref[1:2, :] + be1_ref[...], 0.0)
    z = jnp.dot(z1, We2_ref[...], preferred_element_type=jnp.float32) + be2_ref[...]
    h = h_ref[...]
    A_ref[...] = (jnp.dot(z, Wm_ref[0:D, :], preferred_element_type=jnp.float32)
                  + jnp.dot(h, Wm_ref[D:2 * D, :], preferred_element_type=jnp.float32))
    B_ref[...] = (jnp.dot(z, Wm_ref[2 * D:3 * D, :], preferred_element_type=jnp.float32)
                  + jnp.dot(h, Wm_ref[3 * D:4 * D, :], preferred_element_type=jnp.float32)
                  + bm_ref[...])
    z_ref[...] = z


def _tc1(pos_c, s2_c, h, We1, be1r, We2, be2r, Wm, bmr):
    N, D = h.shape
    grid = N // _R
    row = lambda i: (i, 0)
    full = lambda i: (0, 0)
    return pl.pallas_call(
        _tc1_body,
        grid=(grid,),
        in_specs=[
            pl.BlockSpec((_R, 1), row),
            pl.BlockSpec((_R, 1), row),
            pl.BlockSpec((_R, D), row),
            pl.BlockSpec((2, D), full),
            pl.BlockSpec((1, D), full),
            pl.BlockSpec((D, D), full),
            pl.BlockSpec((1, D), full),
            pl.BlockSpec((4 * D, D), full),
            pl.BlockSpec((1, D), full),
        ],
        out_specs=[pl.BlockSpec((_R, D), row)] * 3,
        out_shape=[jax.ShapeDtypeStruct((N, D), jnp.float32)] * 3,
    )(pos_c, s2_c, h, We1, be1r, We2, be2r, Wm, bmr)


def _sc1_body(A_hbm, B_hbm, srcp, dstp, zeros_hbm, out_hbm,
              src_v, dst_v, a_rows, b_rows, agg_sh, sem):
    c = lax.axis_index("c")
    s_ = lax.axis_index("s")
    w = c * 16 + s_
    rows_per_tile = agg_sh.shape[0] // 16

    pltpu.sync_copy(srcp.at[w], src_v)
    pltpu.sync_copy(dstp.at[w], dst_v)

    base = s_ * rows_per_tile
    pltpu.sync_copy(zeros_hbm.at[pl.ds(base, rows_per_tile)],
                    agg_sh.at[pl.ds(base, rows_per_tile)])
    plsc.subcore_barrier()

    n_chunks = src_v.shape[0]

    def edge_body(e, carry):
        for r in range(8):
            sl = pl.ds(r * 16, 16)
            a_rows[e, sl] = jnp.maximum(a_rows[e, sl] + b_rows[e, sl], 0.0)
        return carry

    def chunk_body(g, carry):
        pltpu.async_copy(A_hbm.at[src_v.at[g]], a_rows, sem).wait()
        pltpu.async_copy(B_hbm.at[dst_v.at[g]], b_rows, sem).wait()
        lax.fori_loop(0, _CHUNK, edge_body, 0, unroll=False)
        pltpu.sync_copy(a_rows, agg_sh.at[dst_v.at[g]], add=True)
        return carry

    lax.fori_loop(0, n_chunks, chunk_body, 0, unroll=False)
    plsc.subcore_barrier()

    pltpu.sync_copy(agg_sh.at[pl.ds(base, rows_per_tile)],
                    out_hbm.at[c, pl.ds(base, rows_per_tile)])


def _sc1(A, B, srcp, dstp, zeros_hbm):
    n_chunks = srcp.shape[1]
    f = pl.kernel(
        _sc1_body,
        mesh=plsc.VectorSubcoreMesh(core_axis_name="c", subcore_axis_name="s"),
        out_type=jax.ShapeDtypeStruct((2, _NP, _D), jnp.float32),
        scratch_types=[
            pltpu.VMEM((n_chunks, _CHUNK), jnp.int32),
            pltpu.VMEM((n_chunks, _CHUNK), jnp.int32),
            pltpu.VMEM((_CHUNK, _D), jnp.float32),
            pltpu.VMEM((_CHUNK, _D), jnp.float32),
            pltpu.VMEM_SHARED((_NP, _D), jnp.float32),
            pltpu.SemaphoreType.DMA,
        ],
    )
    return f(A, B, srcp, dstp, zeros_hbm)


def _tc2_body(z_ref, h_ref, agg0_ref, agg1_ref, Wu_ref, Wa_ref, bu_ref,
              Wd2_ref, bd2_ref, hn_ref, uv_ref):
    D = h_ref.shape[1]
    z = z_ref[...]
    h = h_ref[...]
    agg = agg0_ref[...] + agg1_ref[...]
    hn = jnp.maximum(
        jnp.dot(z, Wu_ref[0:D, :], preferred_element_type=jnp.float32)
        + jnp.dot(h, Wu_ref[D:2 * D, :], preferred_element_type=jnp.float32)
        + jnp.dot(agg, Wa_ref[...], preferred_element_type=jnp.float32)
        + bu_ref[...], 0.0)
    hn_ref[...] = hn
    uv_ref[...] = jnp.dot(hn, Wd2_ref[...], preferred_element_type=jnp.float32) + bd2_ref[...]


def _tc2(z, h, agg0, agg1, Wu, Wa, bur, Wd2, bd2):
    N, D = h.shape
    grid = N // _R
    row = lambda i: (i, 0)
    full = lambda i: (0, 0)
    return pl.pallas_call(
        _tc2_body,
        grid=(grid,),
        in_specs=[
            pl.BlockSpec((_R, D), row),
            pl.BlockSpec((_R, D), row),
            pl.BlockSpec((_R, D), row),
            pl.BlockSpec((_R, D), row),
            pl.BlockSpec((2 * D, D), full),
            pl.BlockSpec((D, D), full),
            pl.BlockSpec((1, D), full),
            pl.BlockSpec((D, 2), full),
            pl.BlockSpec((1, 2), full),
        ],
        out_specs=[pl.BlockSpec((_R, D), row), pl.BlockSpec((_R, 2), row)],
        out_shape=[jax.ShapeDtypeStruct((N, D), jnp.float32),
                   jax.ShapeDtypeStruct((N, 2), jnp.float32)],
    )(z, h, agg0, agg1, Wu, Wa, bur, Wd2, bd2)


def _loss_body(t0_ref, t1_ref, ef_ref, h0_ref, h1_ref, reach_ref, rt_ref,
               nmask_ref, par_ref, pi_ref, lx_ref, lh_ref, lr_ref, lp_ref):
    E = t0_ref.shape[0] * t0_ref.shape[1]
    Nn = 10000.0

    def bce_sum(t, tgt):
        p = 1.0 / (1.0 + jnp.exp(-t))
        p = jnp.clip(p, 1e-7, 1.0 - 1e-7)
        return -jnp.sum(tgt * jnp.log(p) + (1.0 - tgt) * jnp.log(1.0 - p))

    t0 = t0_ref[...]
    t1 = t1_ref[...]
    lx_ref[...] = (bce_sum(t1, ef_ref[...]) / E).reshape(1, 1)
    lh_ref[...] = ((bce_sum(t0, h0_ref[...]) + bce_sum(t1, h1_ref[...])) / E).reshape(1, 1)
    pr = jnp.clip(reach_ref[...], 1e-7, 1.0 - 1e-7)
    rt = rt_ref[...]
    m = nmask_ref[...]
    lr_ref[...] = (-jnp.sum(m * (rt * jnp.log(pr) + (1.0 - rt) * jnp.log(1.0 - pr))) / Nn).reshape(1, 1)
    lp_ref[...] = (1.0 - jnp.sum((par_ref[...] == pi_ref[...]).astype(jnp.float32)) / Nn).reshape(1, 1)


def _tc3(t0, t1, ef, h0, h1, reach_p, rt_p, nmask, par_p, pi_p):
    scal = jax.ShapeDtypeStruct((1, 1), jnp.float32)
    outs = pl.pallas_call(
        _loss_body,
        out_shape=[scal] * 4,
    )(t0, t1, ef, h0, h1, reach_p, rt_p, nmask, par_p, pi_p)
    return tuple(o.reshape(()) for o in outs)


def kernel(pos, s, edge_index, edges, edges_h, reach_h, pi,
           We1, be1, We2, be2, Wm, bm, Wu, Wa, bu, Wd, bd):
    N = pos.shape[0]
    E = edge_index.shape[1]
    D = We1.shape[1]
    T = edges_h.shape[0]
    max_iter = T - 1
    src = edge_index[0]
    dst = edge_index[1]

    npad = _NP - N
    pos_c = jnp.pad(pos, (0, npad)).reshape(_NP, 1)
    be1r = be1.reshape(1, D)
    be2r = be2.reshape(1, D)
    bmr = bm.reshape(1, D)
    bur = bu.reshape(1, D)
    Wd2 = jnp.concatenate([Wd[:D], Wd[D:]], axis=1)
    bd2 = jnp.stack([jnp.zeros((), jnp.float32), bd[0]]).reshape(1, 2)
    THR = jnp.float32(jnp.log(0.4 / 0.6))

    eperw = E // _NW
    n_chunks = -(-eperw // _CHUNK)
    epad = n_chunks * _CHUNK - eperw
    dummy = jnp.full((_NW, epad), _NP - 1, jnp.int32)
    srcp = jnp.concatenate([src.reshape(_NW, eperw), dummy], axis=1) \
              .reshape(_NW, n_chunks, _CHUNK)
    dstp = jnp.concatenate([dst.reshape(_NW, eperw), dummy], axis=1) \
              .reshape(_NW, n_chunks, _CHUNK)
    zeros_hbm = jnp.zeros((_NP, D), jnp.float32)

    h = jnp.zeros((_NP, D), jnp.float32)
    s2 = jnp.pad(s, (0, npad)).reshape(_NP, 1)
    ts = []
    reach = None
    for i in range(max_iter):
        A, B, z = _tc1(pos_c, s2, h, We1, be1r, We2, be2r, Wm, bmr)
        agg2 = _sc1(A, B, srcp, dstp, zeros_hbm)
        h, uv = _tc2(z, h, agg2[0], agg2[1], Wu, Wa, bur, Wd2, bd2)
        t = uv[src, 0] + uv[dst, 1]
        ts.append(t)
        ind = (t >= THR).astype(jnp.float32)
        cnt = (jax.ops.segment_sum(ind, src, num_segments=_NP)
               + jax.ops.segment_sum(ind, dst, num_segments=_NP))
        reach = (cnt > 0.0).astype(jnp.float32)
        s2 = reach.reshape(_NP, 1)

    alpha = jax.nn.sigmoid(ts[-1])
    seg_max = jax.ops.segment_max(alpha, dst, num_segments=N)
    match = jnp.isfinite(seg_max)[dst] & (alpha == seg_max[dst])
    eidx = jnp.where(match, jnp.arange(E, dtype=jnp.int32), jnp.int32(E))
    first = jax.ops.segment_min(eidx, dst, num_segments=N)
    first = jnp.minimum(first, jnp.int32(E))
    parents = jnp.where(first < E, src[jnp.clip(first, 0, E - 1)],
                        jnp.arange(N, dtype=src.dtype))

    Eb = E // D
    t0r = ts[0].reshape(Eb, D)
    t1r = ts[1].reshape(Eb, D)
    ef = edges.astype(jnp.float32).reshape(Eb, D)
    h0 = edges_h[1].astype(jnp.float32).reshape(Eb, D)
    h1 = edges_h[2].astype(jnp.float32).reshape(Eb, D)

    def padf(x, val):
        return jnp.concatenate([x, jnp.full((npad,), val, x.dtype)]).reshape(_NP // D, D)

    reach_p = reach.reshape(_NP // D, D)
    rt_p = padf(reach_h[-1].astype(jnp.float32), 0.5)
    nmask = padf(jnp.ones((N,), jnp.float32), 0.0)
    par_p = padf(parents.astype(jnp.int32), -2)
    pi_p = padf(pi.astype(jnp.int32), -1)
    return _tc3(t0r, t1r, ef, h0, h1, reach_p, rt_p, nmask, par_p, pi_p)


# speedup vs baseline: 5.1477x; 4.6598x over previous
import math

import jax
import jax.numpy as jnp
from jax import lax
from jax.experimental import pallas as pl
from jax.experimental.pallas import tpu as pltpu
from jax.experimental.pallas import tpu_sc as plsc

_NP = 10240
_R = 1024
_D = 128
_NW = 32
_CHUNK = 128
_EPW = 5000
_SLOT = 5120
_SPS = _NP // 16
_THR = float(math.log(0.4 / 0.6))


def _tc1_body(pos_ref, s2_ref, h_ref, We1_ref, be1_ref, We2_ref, be2_ref,
              Wm_ref, bm_ref, A_ref, B_ref, z_ref):
    D = h_ref.shape[1]
    x1 = pos_ref[...]
    x2 = s2_ref[...]
    z1 = jnp.maximum(x1 * We1_ref[0:1, :] + x2 * We1_ref[1:2, :] + be1_ref[...], 0.0)
    z = jnp.dot(z1, We2_ref[...], preferred_element_type=jnp.float32) + be2_ref[...]
    h = h_ref[...]
    A_ref[...] = (jnp.dot(z, Wm_ref[0:D, :], preferred_element_type=jnp.float32)
                  + jnp.dot(h, Wm_ref[D:2 * D, :], preferred_element_type=jnp.float32))
    B_ref[...] = (jnp.dot(z, Wm_ref[2 * D:3 * D, :], preferred_element_type=jnp.float32)
                  + jnp.dot(h, Wm_ref[3 * D:4 * D, :], preferred_element_type=jnp.float32)
                  + bm_ref[...])
    z_ref[...] = z


def _tc1(pos_c, s2_c, h, We1, be1r, We2, be2r, Wm, bmr):
    N, D = h.shape
    grid = N // _R
    row = lambda i: (i, 0)
    full = lambda i: (0, 0)
    return pl.pallas_call(
        _tc1_body,
        grid=(grid,),
        in_specs=[
            pl.BlockSpec((_R, 1), row),
            pl.BlockSpec((_R, 1), row),
            pl.BlockSpec((_R, D), row),
            pl.BlockSpec((2, D), full),
            pl.BlockSpec((1, D), full),
            pl.BlockSpec((D, D), full),
            pl.BlockSpec((1, D), full),
            pl.BlockSpec((4 * D, D), full),
            pl.BlockSpec((1, D), full),
        ],
        out_specs=[pl.BlockSpec((_R, D), row)] * 3,
        out_shape=[jax.ShapeDtypeStruct((N, D), jnp.float32)] * 3,
    )(pos_c, s2_c, h, We1, be1r, We2, be2r, Wm, bmr)


def _sc1_body(A_hbm, B_hbm, srcp, dstp, zeros_hbm, out_hbm,
              src_v, dst_v, a_rows, b_rows, agg_sh, sem):
    c = lax.axis_index("c")
    s_ = lax.axis_index("s")
    w = c * 16 + s_
    rows_per_tile = agg_sh.shape[0] // 16

    pltpu.sync_copy(srcp.at[w], src_v)
    pltpu.sync_copy(dstp.at[w], dst_v)

    base = s_ * rows_per_tile
    pltpu.sync_copy(zeros_hbm.at[pl.ds(base, rows_per_tile)],
                    agg_sh.at[pl.ds(base, rows_per_tile)])
    plsc.subcore_barrier()

    n_chunks = src_v.shape[0]

    def edge_body(e, carry):
        for r in range(8):
            sl = pl.ds(r * 16, 16)
            a_rows[e, sl] = jnp.maximum(a_rows[e, sl] + b_rows[e, sl], 0.0)
        return carry

    def chunk_body(g, carry):
        pltpu.async_copy(A_hbm.at[src_v.at[g]], a_rows, sem).wait()
        pltpu.async_copy(B_hbm.at[dst_v.at[g]], b_rows, sem).wait()
        lax.fori_loop(0, _CHUNK, edge_body, 0, unroll=False)
        pltpu.sync_copy(a_rows, agg_sh.at[dst_v.at[g]], add=True)
        return carry

    lax.fori_loop(0, n_chunks, chunk_body, 0, unroll=False)
    plsc.subcore_barrier()

    pltpu.sync_copy(agg_sh.at[pl.ds(base, rows_per_tile)],
                    out_hbm.at[c, pl.ds(base, rows_per_tile)])


def _sc1(A, B, srcp, dstp, zeros_hbm):
    n_chunks = srcp.shape[1]
    f = pl.kernel(
        _sc1_body,
        mesh=plsc.VectorSubcoreMesh(core_axis_name="c", subcore_axis_name="s"),
        out_type=jax.ShapeDtypeStruct((2, _NP, _D), jnp.float32),
        scratch_types=[
            pltpu.VMEM((n_chunks, _CHUNK), jnp.int32),
            pltpu.VMEM((n_chunks, _CHUNK), jnp.int32),
            pltpu.VMEM((_CHUNK, _D), jnp.float32),
            pltpu.VMEM((_CHUNK, _D), jnp.float32),
            pltpu.VMEM_SHARED((_NP, _D), jnp.float32),
            pltpu.SemaphoreType.DMA,
        ],
    )
    return f(A, B, srcp, dstp, zeros_hbm)


def _sc2_body(u128_hbm, v128_hbm, srcp, dstp, zeros_hbm, t_out, cnt2,
              src_v, dst_v, u_rows, v_rows, cnt_sh, sem):
    c = lax.axis_index("c")
    s_ = lax.axis_index("s")
    w = c * 16 + s_

    pltpu.sync_copy(srcp.at[w], src_v)
    pltpu.sync_copy(dstp.at[w], dst_v)

    base = s_ * _SPS
    pltpu.sync_copy(zeros_hbm.at[pl.ds(base, _SPS)], cnt_sh.at[pl.ds(base, _SPS)])
    plsc.subcore_barrier()

    n_chunks = src_v.shape[0]

    def edge_body(e, carry):
        for r in range(8):
            sl = pl.ds(r * 16, 16)
            t16 = u_rows[e, sl] + v_rows[e, sl]
            u_rows[e, sl] = t16
            v_rows[e, sl] = jnp.where(t16 >= _THR, 1.0, 0.0).astype(jnp.float32)
        return carry

    def chunk_body(g, carry):
        pltpu.async_copy(u128_hbm.at[src_v.at[g]], u_rows, sem).wait()
        pltpu.async_copy(v128_hbm.at[dst_v.at[g]], v_rows, sem).wait()
        lax.fori_loop(0, _CHUNK, edge_body, 0, unroll=False)
        pltpu.sync_copy(u_rows, t_out.at[w, pl.ds(g * _CHUNK, _CHUNK)])
        pltpu.sync_copy(v_rows, cnt_sh.at[src_v.at[g]], add=True)
        pltpu.sync_copy(v_rows, cnt_sh.at[dst_v.at[g]], add=True)
        return carry

    lax.fori_loop(0, n_chunks, chunk_body, 0, unroll=False)
    plsc.subcore_barrier()

    pltpu.sync_copy(cnt_sh.at[pl.ds(base, _SPS)], cnt2.at[c, pl.ds(base, _SPS)])


def _sc2(u128, v128, srcp, dstp, zeros_hbm):
    n_chunks = srcp.shape[1]
    f = pl.kernel(
        _sc2_body,
        mesh=plsc.VectorSubcoreMesh(core_axis_name="c", subcore_axis_name="s"),
        out_type=[jax.ShapeDtypeStruct((_NW, _SLOT, _D), jnp.float32),
                  jax.ShapeDtypeStruct((2, _NP, _D), jnp.float32)],
        scratch_types=[
            pltpu.VMEM((n_chunks, _CHUNK), jnp.int32),
            pltpu.VMEM((n_chunks, _CHUNK), jnp.int32),
            pltpu.VMEM((_CHUNK, _D), jnp.float32),
            pltpu.VMEM((_CHUNK, _D), jnp.float32),
            pltpu.VMEM_SHARED((_NP, _D), jnp.float32),
            pltpu.SemaphoreType.DMA,
        ],
    )
    return f(u128, v128, srcp, dstp, zeros_hbm)


def _sc3_body(m128_hbm, dstp, m_out, dst_v, m_rows, sem):
    c = lax.axis_index("c")
    s_ = lax.axis_index("s")
    w = c * 16 + s_
    pltpu.sync_copy(dstp.at[w], dst_v)
    n_chunks = dst_v.shape[0]

    def chunk_body(g, carry):
        pltpu.async_copy(m128_hbm.at[dst_v.at[g]], m_rows, sem).wait()
        pltpu.sync_copy(m_rows, m_out.at[w, pl.ds(g * _CHUNK, _CHUNK)])
        return carry

    lax.fori_loop(0, n_chunks, chunk_body, 0, unroll=False)


def _sc3(m128, dstp):
    n_chunks = dstp.shape[1]
    f = pl.kernel(
        _sc3_body,
        mesh=plsc.VectorSubcoreMesh(core_axis_name="c", subcore_axis_name="s"),
        out_type=jax.ShapeDtypeStruct((_NW, _SLOT, _D), jnp.float32),
        scratch_types=[
            pltpu.VMEM((n_chunks, _CHUNK), jnp.int32),
            pltpu.VMEM((_CHUNK, _D), jnp.float32),
            pltpu.SemaphoreType.DMA,
        ],
    )
    return f(m128, dstp)


def _tc2_body(z_ref, h_ref, agg0_ref, agg1_ref, Wu_ref, Wa_ref, bu_ref,
              Wd2_ref, bd2_ref, hn_ref, uv_ref):
    D = h_ref.shape[1]
    z = z_ref[...]
    h = h_ref[...]
    agg = agg0_ref[...] + agg1_ref[...]
    hn = jnp.maximum(
        jnp.dot(z, Wu_ref[0:D, :], preferred_element_type=jnp.float32)
        + jnp.dot(h, Wu_ref[D:2 * D, :], preferred_element_type=jnp.float32)
        + jnp.dot(agg, Wa_ref[...], preferred_element_type=jnp.float32)
        + bu_ref[...], 0.0)
    hn_ref[...] = hn
    uv_ref[...] = jnp.dot(hn, Wd2_ref[...], preferred_element_type=jnp.float32) + bd2_ref[...]


def _tc2(z, h, agg0, agg1, Wu, Wa, bur, Wd2, bd2):
    N, D = h.shape
    grid = N // _R
    row = lambda i: (i, 0)
    full = lambda i: (0, 0)
    return pl.pallas_call(
        _tc2_body,
        grid=(grid,),
        in_specs=[
            pl.BlockSpec((_R, D), row),
            pl.BlockSpec((_R, D), row),
            pl.BlockSpec((_R, D), row),
            pl.BlockSpec((_R, D), row),
            pl.BlockSpec((2 * D, D), full),
            pl.BlockSpec((D, D), full),
            pl.BlockSpec((1, D), full),
            pl.BlockSpec((D, 2), full),
            pl.BlockSpec((1, 2), full),
        ],
        out_specs=[pl.BlockSpec((_R, D), row), pl.BlockSpec((_R, 2), row)],
        out_shape=[jax.ShapeDtypeStruct((N, D), jnp.float32),
                   jax.ShapeDtypeStruct((N, 2), jnp.float32)],
    )(z, h, agg0, agg1, Wu, Wa, bur, Wd2, bd2)


def _loss_body(t0_ref, t1_ref, ef_ref, h0_ref, h1_ref, reach_ref, rt_ref,
               nmask_ref, par_ref, pi_ref, lx_ref, lh_ref, lr_ref, lp_ref):
    E = t0_ref.shape[0] * t0_ref.shape[1]
    Nn = 10000.0

    def bce_sum(t, tgt):
        p = 1.0 / (1.0 + jnp.exp(-t))
        p = jnp.clip(p, 1e-7, 1.0 - 1e-7)
        return -jnp.sum(tgt * jnp.log(p) + (1.0 - tgt) * jnp.log(1.0 - p))

    t0 = t0_ref[...]
    t1 = t1_ref[...]
    lx_ref[...] = (bce_sum(t1, ef_ref[...]) / E).reshape(1, 1)
    lh_ref[...] = ((bce_sum(t0, h0_ref[...]) + bce_sum(t1, h1_ref[...])) / E).reshape(1, 1)
    pr = jnp.clip(reach_ref[...], 1e-7, 1.0 - 1e-7)
    rt = rt_ref[...]
    m = nmask_ref[...]
    lr_ref[...] = (-jnp.sum(m * (rt * jnp.log(pr) + (1.0 - rt) * jnp.log(1.0 - pr))) / Nn).reshape(1, 1)
    lp_ref[...] = (1.0 - jnp.sum((par_ref[...] == pi_ref[...]).astype(jnp.float32)) / Nn).reshape(1, 1)


def _tc3(t0, t1, ef, h0, h1, reach_p, rt_p, nmask, par_p, pi_p):
    scal = jax.ShapeDtypeStruct((1, 1), jnp.float32)
    outs = pl.pallas_call(
        _loss_body,
        out_shape=[scal] * 4,
    )(t0, t1, ef, h0, h1, reach_p, rt_p, nmask, par_p, pi_p)
    return tuple(o.reshape(()) for o in outs)


def kernel(pos, s, edge_index, edges, edges_h, reach_h, pi,
           We1, be1, We2, be2, Wm, bm, Wu, Wa, bu, Wd, bd):
    N = pos.shape[0]
    E = edge_index.shape[1]
    D = We1.shape[1]
    T = edges_h.shape[0]
    max_iter = T - 1
    src = edge_index[0]
    dst = edge_index[1]

    npad = _NP - N
    pos_c = jnp.pad(pos, (0, npad)).reshape(_NP, 1)
    be1r = be1.reshape(1, D)
    be2r = be2.reshape(1, D)
    bmr = bm.reshape(1, D)
    bur = bu.reshape(1, D)
    Wd2 = jnp.concatenate([Wd[:D], Wd[D:]], axis=1)
    bd2 = jnp.stack([jnp.zeros((), jnp.float32), bd[0]]).reshape(1, 2)

    eperw = E // _NW
    n_chunks = _SLOT // _CHUNK
    epad = _SLOT - eperw
    dummy = jnp.full((_NW, epad), _NP - 1, jnp.int32)
    srcf = jnp.concatenate([src.reshape(_NW, eperw), dummy], axis=1)
    dstf = jnp.concatenate([dst.reshape(_NW, eperw), dummy], axis=1)
    srcp = srcf.reshape(_NW, n_chunks, _CHUNK)
    dstp = dstf.reshape(_NW, n_chunks, _CHUNK)
    zeros_hbm = jnp.zeros((_NP, D), jnp.float32)
    srcx = jnp.concatenate([src.astype(jnp.int32),
                            jnp.arange(_NP, dtype=jnp.int32)])

    h = jnp.zeros((_NP, D), jnp.float32)
    s2 = jnp.pad(s, (0, npad)).reshape(_NP, 1)
    ts = []
    reach = None
    for i in range(max_iter):
        A, B, z = _tc1(pos_c, s2, h, We1, be1r, We2, be2r, Wm, bmr)
        agg2 = _sc1(A, B, srcp, dstp, zeros_hbm)
        h, uv = _tc2(z, h, agg2[0], agg2[1], Wu, Wa, bur, Wd2, bd2)
        u128 = jnp.broadcast_to(uv[:, 0:1], (_NP, _D))
        v128 = jnp.broadcast_to(uv[:, 1:2], (_NP, _D))
        t_out, cnt2 = _sc2(u128, v128, srcp, dstp, zeros_hbm)
        ts.append(t_out[:, :eperw, 0].reshape(E))
        reach = jnp.where(cnt2[0, :, 0] + cnt2[1, :, 0] > 0.0, 1.0, 0.0)
        s2 = reach.reshape(_NP, 1)

    alpha = jax.nn.sigmoid(ts[-1])
    seg_max = jax.ops.segment_max(alpha, dst, num_segments=_NP)
    m128 = jnp.broadcast_to(seg_max.reshape(_NP, 1), (_NP, _D))
    m_e = _sc3(m128, dstp)[:, :eperw, 0].reshape(E)
    match = alpha == m_e
    eidx = jnp.where(match, jnp.arange(E, dtype=jnp.int32), jnp.int32(E + _NP))
    first = jax.ops.segment_min(eidx, dst, num_segments=_NP)
    own = jnp.arange(_NP, dtype=jnp.int32)
    ext_idx = jnp.where(first < E, first, E + own)
    parents = jnp.take(srcx, ext_idx)[:N]

    Eb = E // D
    t0r = ts[0].reshape(Eb, D)
    t1r = ts[1].reshape(Eb, D)
    ef = edges.astype(jnp.float32).reshape(Eb, D)
    h0 = edges_h[1].astype(jnp.float32).reshape(Eb, D)
    h1 = edges_h[2].astype(jnp.float32).reshape(Eb, D)

    def padf(x, val):
        return jnp.concatenate([x, jnp.full((npad,), val, x.dtype)]).reshape(_NP // D, D)

    reach_p = reach.reshape(_NP // D, D)
    rt_p = padf(reach_h[-1].astype(jnp.float32), 0.5)
    nmask = padf(jnp.ones((N,), jnp.float32), 0.0)
    par_p = padf(parents.astype(jnp.int32), -2)
    pi_p = padf(pi.astype(jnp.int32), -1)
    return _tc3(t0r, t1r, ef, h0, h1, reach_p, rt_p, nmask, par_p, pi_p)


# speedup vs baseline: 6.3366x; 1.2310x over previous
import math

import jax
import jax.numpy as jnp
from jax import lax
from jax.experimental import pallas as pl
from jax.experimental.pallas import tpu as pltpu
from jax.experimental.pallas import tpu_sc as plsc

_NP = 10240
_R = 1024
_D = 128
_NW = 32
_CHUNK = 128
_EPW = 5000
_SLOT = 5120
_SPS = _NP // 16
_THR = float(math.log(0.4 / 0.6))


def _tc1_body(pos_ref, s2_ref, h_ref, We1_ref, be1_ref, We2_ref, be2_ref,
              Wm_ref, bm_ref, A_ref, B_ref, z_ref):
    D = h_ref.shape[1]
    x1 = pos_ref[...]
    x2 = s2_ref[...]
    z1 = jnp.maximum(x1 * We1_ref[0:1, :] + x2 * We1_ref[1:2, :] + be1_ref[...], 0.0)
    z = jnp.dot(z1, We2_ref[...], preferred_element_type=jnp.float32) + be2_ref[...]
    h = h_ref[...]
    A_ref[...] = (jnp.dot(z, Wm_ref[0:D, :], preferred_element_type=jnp.float32)
                  + jnp.dot(h, Wm_ref[D:2 * D, :], preferred_element_type=jnp.float32))
    B_ref[...] = (jnp.dot(z, Wm_ref[2 * D:3 * D, :], preferred_element_type=jnp.float32)
                  + jnp.dot(h, Wm_ref[3 * D:4 * D, :], preferred_element_type=jnp.float32)
                  + bm_ref[...])
    z_ref[...] = z


def _tc1(pos_c, s2_c, h, We1, be1r, We2, be2r, Wm, bmr):
    N, D = h.shape
    grid = N // _R
    row = lambda i: (i, 0)
    full = lambda i: (0, 0)
    return pl.pallas_call(
        _tc1_body,
        grid=(grid,),
        in_specs=[
            pl.BlockSpec((_R, 1), row),
            pl.BlockSpec((_R, 1), row),
            pl.BlockSpec((_R, D), row),
            pl.BlockSpec((2, D), full),
            pl.BlockSpec((1, D), full),
            pl.BlockSpec((D, D), full),
            pl.BlockSpec((1, D), full),
            pl.BlockSpec((4 * D, D), full),
            pl.BlockSpec((1, D), full),
        ],
        out_specs=[pl.BlockSpec((_R, D), row)] * 3,
        out_shape=[jax.ShapeDtypeStruct((N, D), jnp.float32)] * 3,
    )(pos_c, s2_c, h, We1, be1r, We2, be2r, Wm, bmr)


def _sc1_body(A_hbm, B_hbm, srcp, dstp, zeros_hbm, out_hbm,
              src_v, dst_v, a_rows, b_rows, agg_sh, sem, sem2):
    c = lax.axis_index("c")
    s_ = lax.axis_index("s")
    w = c * 16 + s_
    rows_per_tile = agg_sh.shape[0] // 16

    pltpu.sync_copy(srcp.at[w], src_v)
    pltpu.sync_copy(dstp.at[w], dst_v)

    base = s_ * rows_per_tile
    pltpu.sync_copy(zeros_hbm.at[pl.ds(base, rows_per_tile)],
                    agg_sh.at[pl.ds(base, rows_per_tile)])
    plsc.subcore_barrier()

    n_chunks = src_v.shape[0]

    def edge_body(e, carry):
        for r in range(8):
            sl = pl.ds(r * 16, 16)
            a_rows[e, sl] = jnp.maximum(a_rows[e, sl] + b_rows[e, sl], 0.0)
        return carry

    def chunk_body(g, carry):
        cp_a = pltpu.async_copy(A_hbm.at[src_v.at[g]], a_rows, sem)
        cp_b = pltpu.async_copy(B_hbm.at[dst_v.at[g]], b_rows, sem2)
        cp_a.wait()
        cp_b.wait()
        lax.fori_loop(0, _CHUNK, edge_body, 0, unroll=False)
        pltpu.sync_copy(a_rows, agg_sh.at[dst_v.at[g]], add=True)
        return carry

    lax.fori_loop(0, n_chunks, chunk_body, 0, unroll=False)
    plsc.subcore_barrier()

    pltpu.sync_copy(agg_sh.at[pl.ds(base, rows_per_tile)],
                    out_hbm.at[c, pl.ds(base, rows_per_tile)])


def _sc1(A, B, srcp, dstp, zeros_hbm):
    n_chunks = srcp.shape[1]
    f = pl.kernel(
        _sc1_body,
        mesh=plsc.VectorSubcoreMesh(core_axis_name="c", subcore_axis_name="s"),
        out_type=jax.ShapeDtypeStruct((2, _NP, _D), jnp.float32),
        scratch_types=[
            pltpu.VMEM((n_chunks, _CHUNK), jnp.int32),
            pltpu.VMEM((n_chunks, _CHUNK), jnp.int32),
            pltpu.VMEM((_CHUNK, _D), jnp.float32),
            pltpu.VMEM((_CHUNK, _D), jnp.float32),
            pltpu.VMEM_SHARED((_NP, _D), jnp.float32),
            pltpu.SemaphoreType.DMA,
            pltpu.SemaphoreType.DMA,
        ],
    )
    return f(A, B, srcp, dstp, zeros_hbm)


def _sc2_body(u128_hbm, v128_hbm, srcp, dstp, zeros_hbm, t_out, cnt2,
              src_v, dst_v, u_rows, v_rows, cnt_sh, sem, sem2):
    c = lax.axis_index("c")
    s_ = lax.axis_index("s")
    w = c * 16 + s_

    pltpu.sync_copy(srcp.at[w], src_v)
    pltpu.sync_copy(dstp.at[w], dst_v)

    base = s_ * _SPS
    pltpu.sync_copy(zeros_hbm.at[pl.ds(base, _SPS)], cnt_sh.at[pl.ds(base, _SPS)])
    plsc.subcore_barrier()

    n_chunks = src_v.shape[0]

    def edge_body(e, carry):
        for r in range(8):
            sl = pl.ds(r * 16, 16)
            t16 = u_rows[e, sl] + v_rows[e, sl]
            u_rows[e, sl] = t16
            v_rows[e, sl] = jnp.where(t16 >= _THR, 1.0, 0.0).astype(jnp.float32)
        return carry

    def chunk_body(g, carry):
        cp_u = pltpu.async_copy(u128_hbm.at[src_v.at[g]], u_rows, sem)
        cp_v = pltpu.async_copy(v128_hbm.at[dst_v.at[g]], v_rows, sem2)
        cp_u.wait()
        cp_v.wait()
        lax.fori_loop(0, _CHUNK, edge_body, 0, unroll=False)
        pltpu.sync_copy(u_rows, t_out.at[w, pl.ds(g * _CHUNK, _CHUNK)])
        pltpu.sync_copy(v_rows, cnt_sh.at[src_v.at[g]], add=True)
        pltpu.sync_copy(v_rows, cnt_sh.at[dst_v.at[g]], add=True)
        return carry

    lax.fori_loop(0, n_chunks, chunk_body, 0, unroll=False)
    plsc.subcore_barrier()

    pltpu.sync_copy(cnt_sh.at[pl.ds(base, _SPS)], cnt2.at[c, pl.ds(base, _SPS)])


def _sc2(u128, v128, srcp, dstp, zeros_hbm):
    n_chunks = srcp.shape[1]
    f = pl.kernel(
        _sc2_body,
        mesh=plsc.VectorSubcoreMesh(core_axis_name="c", subcore_axis_name="s"),
        out_type=[jax.ShapeDtypeStruct((_NW, _SLOT, _D), jnp.float32),
                  jax.ShapeDtypeStruct((2, _NP, _D), jnp.float32)],
        scratch_types=[
            pltpu.VMEM((n_chunks, _CHUNK), jnp.int32),
            pltpu.VMEM((n_chunks, _CHUNK), jnp.int32),
            pltpu.VMEM((_CHUNK, _D), jnp.float32),
            pltpu.VMEM((_CHUNK, _D), jnp.float32),
            pltpu.VMEM_SHARED((_NP, _D), jnp.float32),
            pltpu.SemaphoreType.DMA,
            pltpu.SemaphoreType.DMA,
        ],
    )
    return f(u128, v128, srcp, dstp, zeros_hbm)


def _sc3_body(m128_hbm, dstp, m_out, dst_v, m_rows, sem):
    c = lax.axis_index("c")
    s_ = lax.axis_index("s")
    w = c * 16 + s_
    pltpu.sync_copy(dstp.at[w], dst_v)
    n_chunks = dst_v.shape[0]

    def chunk_body(g, carry):
        pltpu.async_copy(m128_hbm.at[dst_v.at[g]], m_rows, sem).wait()
        pltpu.sync_copy(m_rows, m_out.at[w, pl.ds(g * _CHUNK, _CHUNK)])
        return carry

    lax.fori_loop(0, n_chunks, chunk_body, 0, unroll=False)


def _sc3(m128, dstp):
    n_chunks = dstp.shape[1]
    f = pl.kernel(
        _sc3_body,
        mesh=plsc.VectorSubcoreMesh(core_axis_name="c", subcore_axis_name="s"),
        out_type=jax.ShapeDtypeStruct((_NW, _SLOT, _D), jnp.float32),
        scratch_types=[
            pltpu.VMEM((n_chunks, _CHUNK), jnp.int32),
            pltpu.VMEM((_CHUNK, _D), jnp.float32),
            pltpu.SemaphoreType.DMA,
        ],
    )
    return f(m128, dstp)


def _tc2_body(z_ref, h_ref, agg0_ref, agg1_ref, Wu_ref, Wa_ref, bu_ref,
              Wd2_ref, bd2_ref, hn_ref, uv_ref):
    D = h_ref.shape[1]
    z = z_ref[...]
    h = h_ref[...]
    agg = agg0_ref[...] + agg1_ref[...]
    hn = jnp.maximum(
        jnp.dot(z, Wu_ref[0:D, :], preferred_element_type=jnp.float32)
        + jnp.dot(h, Wu_ref[D:2 * D, :], preferred_element_type=jnp.float32)
        + jnp.dot(agg, Wa_ref[...], preferred_element_type=jnp.float32)
        + bu_ref[...], 0.0)
    hn_ref[...] = hn
    uv_ref[...] = jnp.dot(hn, Wd2_ref[...], preferred_element_type=jnp.float32) + bd2_ref[...]


def _tc2(z, h, agg0, agg1, Wu, Wa, bur, Wd2, bd2):
    N, D = h.shape
    grid = N // _R
    row = lambda i: (i, 0)
    full = lambda i: (0, 0)
    return pl.pallas_call(
        _tc2_body,
        grid=(grid,),
        in_specs=[
            pl.BlockSpec((_R, D), row),
            pl.BlockSpec((_R, D), row),
            pl.BlockSpec((_R, D), row),
            pl.BlockSpec((_R, D), row),
            pl.BlockSpec((2 * D, D), full),
            pl.BlockSpec((D, D), full),
            pl.BlockSpec((1, D), full),
            pl.BlockSpec((D, 2), full),
            pl.BlockSpec((1, 2), full),
        ],
        out_specs=[pl.BlockSpec((_R, D), row), pl.BlockSpec((_R, 2), row)],
        out_shape=[jax.ShapeDtypeStruct((N, D), jnp.float32),
                   jax.ShapeDtypeStruct((N, 2), jnp.float32)],
    )(z, h, agg0, agg1, Wu, Wa, bur, Wd2, bd2)


def _loss_body(t0_ref, t1_ref, ef_ref, h0_ref, h1_ref, reach_ref, rt_ref,
               nmask_ref, par_ref, pi_ref, lx_ref, lh_ref, lr_ref, lp_ref):
    E = t0_ref.shape[0] * t0_ref.shape[1]
    Nn = 10000.0

    def bce_sum(t, tgt):
        p = 1.0 / (1.0 + jnp.exp(-t))
        p = jnp.clip(p, 1e-7, 1.0 - 1e-7)
        return -jnp.sum(tgt * jnp.log(p) + (1.0 - tgt) * jnp.log(1.0 - p))

    t0 = t0_ref[...]
    t1 = t1_ref[...]
    lx_ref[...] = (bce_sum(t1, ef_ref[...]) / E).reshape(1, 1)
    lh_ref[...] = ((bce_sum(t0, h0_ref[...]) + bce_sum(t1, h1_ref[...])) / E).reshape(1, 1)
    pr = jnp.clip(reach_ref[...], 1e-7, 1.0 - 1e-7)
    rt = rt_ref[...]
    m = nmask_ref[...]
    lr_ref[...] = (-jnp.sum(m * (rt * jnp.log(pr) + (1.0 - rt) * jnp.log(1.0 - pr))) / Nn).reshape(1, 1)
    lp_ref[...] = (1.0 - jnp.sum((par_ref[...] == pi_ref[...]).astype(jnp.float32)) / Nn).reshape(1, 1)


def _tc3(t0, t1, ef, h0, h1, reach_p, rt_p, nmask, par_p, pi_p):
    scal = jax.ShapeDtypeStruct((1, 1), jnp.float32)
    outs = pl.pallas_call(
        _loss_body,
        out_shape=[scal] * 4,
    )(t0, t1, ef, h0, h1, reach_p, rt_p, nmask, par_p, pi_p)
    return tuple(o.reshape(()) for o in outs)


def kernel(pos, s, edge_index, edges, edges_h, reach_h, pi,
           We1, be1, We2, be2, Wm, bm, Wu, Wa, bu, Wd, bd):
    N = pos.shape[0]
    E = edge_index.shape[1]
    D = We1.shape[1]
    T = edges_h.shape[0]
    max_iter = T - 1
    src = edge_index[0]
    dst = edge_index[1]

    npad = _NP - N
    pos_c = jnp.pad(pos, (0, npad)).reshape(_NP, 1)
    be1r = be1.reshape(1, D)
    be2r = be2.reshape(1, D)
    bmr = bm.reshape(1, D)
    bur = bu.reshape(1, D)
    Wd2 = jnp.concatenate([Wd[:D], Wd[D:]], axis=1)
    bd2 = jnp.stack([jnp.zeros((), jnp.float32), bd[0]]).reshape(1, 2)

    eperw = E // _NW
    n_chunks = _SLOT // _CHUNK
    epad = _SLOT - eperw
    dummy = jnp.full((_NW, epad), _NP - 1, jnp.int32)
    srcf = jnp.concatenate([src.reshape(_NW, eperw), dummy], axis=1)
    dstf = jnp.concatenate([dst.reshape(_NW, eperw), dummy], axis=1)
    srcp = srcf.reshape(_NW, n_chunks, _CHUNK)
    dstp = dstf.reshape(_NW, n_chunks, _CHUNK)
    zeros_hbm = jnp.zeros((_NP, D), jnp.float32)
    srcx = jnp.concatenate([src.astype(jnp.int32),
                            jnp.arange(_NP, dtype=jnp.int32)])

    h = jnp.zeros((_NP, D), jnp.float32)
    s2 = jnp.pad(s, (0, npad)).reshape(_NP, 1)
    ts = []
    reach = None
    for i in range(max_iter):
        A, B, z = _tc1(pos_c, s2, h, We1, be1r, We2, be2r, Wm, bmr)
        agg2 = _sc1(A, B, srcp, dstp, zeros_hbm)
        h, uv = _tc2(z, h, agg2[0], agg2[1], Wu, Wa, bur, Wd2, bd2)
        u128 = jnp.broadcast_to(uv[:, 0:1], (_NP, _D))
        v128 = jnp.broadcast_to(uv[:, 1:2], (_NP, _D))
        t_out, cnt2 = _sc2(u128, v128, srcp, dstp, zeros_hbm)
        ts.append(t_out[:, :eperw, 0].reshape(E))
        reach = jnp.where(cnt2[0, :, 0] + cnt2[1, :, 0] > 0.0, 1.0, 0.0)
        s2 = reach.reshape(_NP, 1)

    alpha = jax.nn.sigmoid(ts[-1])
    seg_max = jax.ops.segment_max(alpha, dst, num_segments=_NP)
    m128 = jnp.broadcast_to(seg_max.reshape(_NP, 1), (_NP, _D))
    m_e = _sc3(m128, dstp)[:, :eperw, 0].reshape(E)
    match = alpha == m_e
    eidx = jnp.where(match, jnp.arange(E, dtype=jnp.int32), jnp.int32(E + _NP))
    first = jax.ops.segment_min(eidx, dst, num_segments=_NP)
    own = jnp.arange(_NP, dtype=jnp.int32)
    ext_idx = jnp.where(first < E, first, E + own)
    parents = jnp.take(srcx, ext_idx)[:N]

    Eb = E // D
    t0r = ts[0].reshape(Eb, D)
    t1r = ts[1].reshape(Eb, D)
    ef = edges.astype(jnp.float32).reshape(Eb, D)
    h0 = edges_h[1].astype(jnp.float32).reshape(Eb, D)
    h1 = edges_h[2].astype(jnp.float32).reshape(Eb, D)

    def padf(x, val):
        return jnp.concatenate([x, jnp.full((npad,), val, x.dtype)]).reshape(_NP // D, D)

    reach_p = reach.reshape(_NP // D, D)
    rt_p = padf(reach_h[-1].astype(jnp.float32), 0.5)
    nmask = padf(jnp.ones((N,), jnp.float32), 0.0)
    par_p = padf(parents.astype(jnp.int32), -2)
    pi_p = padf(pi.astype(jnp.int32), -1)
    return _tc3(t0r, t1r, ef, h0, h1, reach_p, rt_p, nmask, par_p, pi_p)


# speedup vs baseline: 6.4254x; 1.0140x over previous
import math

import jax
import jax.numpy as jnp
from jax import lax
from jax.experimental import pallas as pl
from jax.experimental.pallas import tpu as pltpu
from jax.experimental.pallas import tpu_sc as plsc

_NP = 10240
_R = 1024
_D = 128
_NW = 32
_CHUNK = 128
_EPW = 5000
_SLOT = 5120
_SPS = _NP // 16
_THR = float(math.log(0.4 / 0.6))


def _tc1_body(pos_ref, s2_ref, h_ref, We1_ref, be1_ref, We2_ref, be2_ref,
              Wm_ref, bm_ref, A_ref, B_ref, z_ref):
    D = h_ref.shape[1]
    x1 = pos_ref[...]
    x2 = s2_ref[...]
    z1 = jnp.maximum(x1 * We1_ref[0:1, :] + x2 * We1_ref[1:2, :] + be1_ref[...], 0.0)
    z = jnp.dot(z1, We2_ref[...], preferred_element_type=jnp.float32) + be2_ref[...]
    h = h_ref[...]
    A_ref[...] = (jnp.dot(z, Wm_ref[0:D, :], preferred_element_type=jnp.float32)
                  + jnp.dot(h, Wm_ref[D:2 * D, :], preferred_element_type=jnp.float32))
    B_ref[...] = (jnp.dot(z, Wm_ref[2 * D:3 * D, :], preferred_element_type=jnp.float32)
                  + jnp.dot(h, Wm_ref[3 * D:4 * D, :], preferred_element_type=jnp.float32)
                  + bm_ref[...])
    z_ref[...] = z


def _tc1(pos_c, s2_c, h, We1, be1r, We2, be2r, Wm, bmr):
    N, D = h.shape
    grid = N // _R
    row = lambda i: (i, 0)
    full = lambda i: (0, 0)
    return pl.pallas_call(
        _tc1_body,
        grid=(grid,),
        in_specs=[
            pl.BlockSpec((_R, 1), row),
            pl.BlockSpec((_R, 1), row),
            pl.BlockSpec((_R, D), row),
            pl.BlockSpec((2, D), full),
            pl.BlockSpec((1, D), full),
            pl.BlockSpec((D, D), full),
            pl.BlockSpec((1, D), full),
            pl.BlockSpec((4 * D, D), full),
            pl.BlockSpec((1, D), full),
        ],
        out_specs=[pl.BlockSpec((_R, D), row)] * 3,
        out_shape=[jax.ShapeDtypeStruct((N, D), jnp.float32)] * 3,
    )(pos_c, s2_c, h, We1, be1r, We2, be2r, Wm, bmr)


def _sc1_body(A_hbm, B_hbm, srcp, dstp, zeros_hbm, out_hbm,
              src_v, dst_v, a_rows, b_rows, agg_sh, sem, sem2):
    c = lax.axis_index("c")
    s_ = lax.axis_index("s")
    w = c * 16 + s_
    rows_per_tile = agg_sh.shape[0] // 16

    pltpu.sync_copy(srcp.at[w], src_v)
    pltpu.sync_copy(dstp.at[w], dst_v)

    base = s_ * rows_per_tile
    pltpu.sync_copy(zeros_hbm.at[pl.ds(base, rows_per_tile)],
                    agg_sh.at[pl.ds(base, rows_per_tile)])
    plsc.subcore_barrier()

    n_chunks = src_v.shape[0]

    def edge_body(e, carry):
        for r in range(8):
            sl = pl.ds(r * 16, 16)
            a_rows[e, sl] = jnp.maximum(a_rows[e, sl] + b_rows[e, sl], 0.0)
        return carry

    def chunk_body(g, carry):
        cp_a = pltpu.async_copy(A_hbm.at[src_v.at[g]], a_rows, sem)
        cp_b = pltpu.async_copy(B_hbm.at[dst_v.at[g]], b_rows, sem2)
        cp_a.wait()
        cp_b.wait()
        lax.fori_loop(0, _CHUNK, edge_body, 0, unroll=False)
        pltpu.sync_copy(a_rows, agg_sh.at[dst_v.at[g]], add=True)
        return carry

    lax.fori_loop(0, n_chunks, chunk_body, 0, unroll=False)
    plsc.subcore_barrier()

    pltpu.sync_copy(agg_sh.at[pl.ds(base, rows_per_tile)],
                    out_hbm.at[c, pl.ds(base, rows_per_tile)])


def _sc1(A, B, srcp, dstp, zeros_hbm):
    n_chunks = srcp.shape[1]
    f = pl.kernel(
        _sc1_body,
        mesh=plsc.VectorSubcoreMesh(core_axis_name="c", subcore_axis_name="s"),
        out_type=jax.ShapeDtypeStruct((2, _NP, _D), jnp.float32),
        scratch_types=[
            pltpu.VMEM((n_chunks, _CHUNK), jnp.int32),
            pltpu.VMEM((n_chunks, _CHUNK), jnp.int32),
            pltpu.VMEM((_CHUNK, _D), jnp.float32),
            pltpu.VMEM((_CHUNK, _D), jnp.float32),
            pltpu.VMEM_SHARED((_NP, _D), jnp.float32),
            pltpu.SemaphoreType.DMA,
            pltpu.SemaphoreType.DMA,
        ],
    )
    return f(A, B, srcp, dstp, zeros_hbm)


def _sc2_body(u128_hbm, v128_hbm, srcp, dstp, zeros_hbm, t_out, cnt2,
              src_v, dst_v, u_rows, v_rows, cnt_sh, sem, sem2, sem3):
    c = lax.axis_index("c")
    s_ = lax.axis_index("s")
    w = c * 16 + s_

    pltpu.sync_copy(srcp.at[w], src_v)
    pltpu.sync_copy(dstp.at[w], dst_v)

    base = s_ * _SPS
    pltpu.sync_copy(zeros_hbm.at[pl.ds(base, _SPS)], cnt_sh.at[pl.ds(base, _SPS)])
    plsc.subcore_barrier()

    n_chunks = src_v.shape[0]

    def edge_body(e, carry):
        for r in range(8):
            sl = pl.ds(r * 16, 16)
            t16 = u_rows[e, sl] + v_rows[e, sl]
            u_rows[e, sl] = t16
            v_rows[e, sl] = jnp.where(t16 >= _THR, 1.0, 0.0).astype(jnp.float32)
        return carry

    def chunk_body(g, carry):
        cp_u = pltpu.async_copy(u128_hbm.at[src_v.at[g]], u_rows, sem)
        cp_v = pltpu.async_copy(v128_hbm.at[dst_v.at[g]], v_rows, sem2)
        cp_u.wait()
        cp_v.wait()
        lax.fori_loop(0, _CHUNK, edge_body, 0, unroll=False)
        cp_t = pltpu.async_copy(u_rows, t_out.at[w, pl.ds(g * _CHUNK, _CHUNK)], sem3)
        pltpu.sync_copy(v_rows, cnt_sh.at[src_v.at[g]], add=True)
        pltpu.sync_copy(v_rows, cnt_sh.at[dst_v.at[g]], add=True)
        cp_t.wait()
        return carry

    lax.fori_loop(0, n_chunks, chunk_body, 0, unroll=False)
    plsc.subcore_barrier()

    pltpu.sync_copy(cnt_sh.at[pl.ds(base, _SPS)], cnt2.at[c, pl.ds(base, _SPS)])


def _sc2(u128, v128, srcp, dstp, zeros_hbm):
    n_chunks = srcp.shape[1]
    f = pl.kernel(
        _sc2_body,
        mesh=plsc.VectorSubcoreMesh(core_axis_name="c", subcore_axis_name="s"),
        out_type=[jax.ShapeDtypeStruct((_NW, _SLOT, _D), jnp.float32),
                  jax.ShapeDtypeStruct((2, _NP, _D), jnp.float32)],
        scratch_types=[
            pltpu.VMEM((n_chunks, _CHUNK), jnp.int32),
            pltpu.VMEM((n_chunks, _CHUNK), jnp.int32),
            pltpu.VMEM((_CHUNK, _D), jnp.float32),
            pltpu.VMEM((_CHUNK, _D), jnp.float32),
            pltpu.VMEM_SHARED((_NP, _D), jnp.float32),
            pltpu.SemaphoreType.DMA,
            pltpu.SemaphoreType.DMA,
            pltpu.SemaphoreType.DMA,
        ],
    )
    return f(u128, v128, srcp, dstp, zeros_hbm)


def _sc3_body(m128_hbm, dstp, m_out, dst_v, m_rows, sem):
    c = lax.axis_index("c")
    s_ = lax.axis_index("s")
    w = c * 16 + s_
    pltpu.sync_copy(dstp.at[w], dst_v)
    n_chunks = dst_v.shape[0]

    def chunk_body(g, carry):
        pltpu.async_copy(m128_hbm.at[dst_v.at[g]], m_rows, sem).wait()
        pltpu.sync_copy(m_rows, m_out.at[w, pl.ds(g * _CHUNK, _CHUNK)])
        return carry

    lax.fori_loop(0, n_chunks, chunk_body, 0, unroll=False)


def _sc3(m128, dstp):
    n_chunks = dstp.shape[1]
    f = pl.kernel(
        _sc3_body,
        mesh=plsc.VectorSubcoreMesh(core_axis_name="c", subcore_axis_name="s"),
        out_type=jax.ShapeDtypeStruct((_NW, _SLOT, _D), jnp.float32),
        scratch_types=[
            pltpu.VMEM((n_chunks, _CHUNK), jnp.int32),
            pltpu.VMEM((_CHUNK, _D), jnp.float32),
            pltpu.SemaphoreType.DMA,
        ],
    )
    return f(m128, dstp)


def _tc2_body(z_ref, h_ref, agg0_ref, agg1_ref, Wu_ref, Wa_ref, bu_ref,
              Wd2_ref, bd2_ref, hn_ref, uv_ref):
    D = h_ref.shape[1]
    z = z_ref[...]
    h = h_ref[...]
    agg = agg0_ref[...] + agg1_ref[...]
    hn = jnp.maximum(
        jnp.dot(z, Wu_ref[0:D, :], preferred_element_type=jnp.float32)
        + jnp.dot(h, Wu_ref[D:2 * D, :], preferred_element_type=jnp.float32)
        + jnp.dot(agg, Wa_ref[...], preferred_element_type=jnp.float32)
        + bu_ref[...], 0.0)
    hn_ref[...] = hn
    uv_ref[...] = jnp.dot(hn, Wd2_ref[...], preferred_element_type=jnp.float32) + bd2_ref[...]


def _tc2(z, h, agg0, agg1, Wu, Wa, bur, Wd2, bd2):
    N, D = h.shape
    grid = N // _R
    row = lambda i: (i, 0)
    full = lambda i: (0, 0)
    return pl.pallas_call(
        _tc2_body,
        grid=(grid,),
        in_specs=[
            pl.BlockSpec((_R, D), row),
            pl.BlockSpec((_R, D), row),
            pl.BlockSpec((_R, D), row),
            pl.BlockSpec((_R, D), row),
            pl.BlockSpec((2 * D, D), full),
            pl.BlockSpec((D, D), full),
            pl.BlockSpec((1, D), full),
            pl.BlockSpec((D, 2), full),
            pl.BlockSpec((1, 2), full),
        ],
        out_specs=[pl.BlockSpec((_R, D), row), pl.BlockSpec((_R, 2), row)],
        out_shape=[jax.ShapeDtypeStruct((N, D), jnp.float32),
                   jax.ShapeDtypeStruct((N, 2), jnp.float32)],
    )(z, h, agg0, agg1, Wu, Wa, bur, Wd2, bd2)


def _loss_body(t0_ref, t1_ref, ef_ref, h0_ref, h1_ref, reach_ref, rt_ref,
               nmask_ref, par_ref, pi_ref, lx_ref, lh_ref, lr_ref, lp_ref):
    E = t0_ref.shape[0] * t0_ref.shape[1]
    Nn = 10000.0

    def bce_sum(t, tgt):
        p = 1.0 / (1.0 + jnp.exp(-t))
        p = jnp.clip(p, 1e-7, 1.0 - 1e-7)
        return -jnp.sum(tgt * jnp.log(p) + (1.0 - tgt) * jnp.log(1.0 - p))

    t0 = t0_ref[...]
    t1 = t1_ref[...]
    lx_ref[...] = (bce_sum(t1, ef_ref[...]) / E).reshape(1, 1)
    lh_ref[...] = ((bce_sum(t0, h0_ref[...]) + bce_sum(t1, h1_ref[...])) / E).reshape(1, 1)
    pr = jnp.clip(reach_ref[...], 1e-7, 1.0 - 1e-7)
    rt = rt_ref[...]
    m = nmask_ref[...]
    lr_ref[...] = (-jnp.sum(m * (rt * jnp.log(pr) + (1.0 - rt) * jnp.log(1.0 - pr))) / Nn).reshape(1, 1)
    lp_ref[...] = (1.0 - jnp.sum((par_ref[...] == pi_ref[...]).astype(jnp.float32)) / Nn).reshape(1, 1)


def _tc3(t0, t1, ef, h0, h1, reach_p, rt_p, nmask, par_p, pi_p):
    scal = jax.ShapeDtypeStruct((1, 1), jnp.float32)
    outs = pl.pallas_call(
        _loss_body,
        out_shape=[scal] * 4,
    )(t0, t1, ef, h0, h1, reach_p, rt_p, nmask, par_p, pi_p)
    return tuple(o.reshape(()) for o in outs)


def kernel(pos, s, edge_index, edges, edges_h, reach_h, pi,
           We1, be1, We2, be2, Wm, bm, Wu, Wa, bu, Wd, bd):
    N = pos.shape[0]
    E = edge_index.shape[1]
    D = We1.shape[1]
    T = edges_h.shape[0]
    max_iter = T - 1
    src = edge_index[0]
    dst = edge_index[1]

    npad = _NP - N
    pos_c = jnp.pad(pos, (0, npad)).reshape(_NP, 1)
    be1r = be1.reshape(1, D)
    be2r = be2.reshape(1, D)
    bmr = bm.reshape(1, D)
    bur = bu.reshape(1, D)
    Wd2 = jnp.concatenate([Wd[:D], Wd[D:]], axis=1)
    bd2 = jnp.stack([jnp.zeros((), jnp.float32), bd[0]]).reshape(1, 2)

    eperw = E // _NW
    n_chunks = _SLOT // _CHUNK
    epad = _SLOT - eperw
    dummy = jnp.full((_NW, epad), _NP - 1, jnp.int32)
    srcf = jnp.concatenate([src.reshape(_NW, eperw), dummy], axis=1)
    dstf = jnp.concatenate([dst.reshape(_NW, eperw), dummy], axis=1)
    srcp = srcf.reshape(_NW, n_chunks, _CHUNK)
    dstp = dstf.reshape(_NW, n_chunks, _CHUNK)
    zeros_hbm = jnp.zeros((_NP, D), jnp.float32)
    srcx = jnp.concatenate([src.astype(jnp.int32),
                            jnp.arange(_NP, dtype=jnp.int32)])

    h = jnp.zeros((_NP, D), jnp.float32)
    s2 = jnp.pad(s, (0, npad)).reshape(_NP, 1)
    ts = []
    reach = None
    for i in range(max_iter):
        A, B, z = _tc1(pos_c, s2, h, We1, be1r, We2, be2r, Wm, bmr)
        agg2 = _sc1(A, B, srcp, dstp, zeros_hbm)
        h, uv = _tc2(z, h, agg2[0], agg2[1], Wu, Wa, bur, Wd2, bd2)
        u128 = jnp.broadcast_to(uv[:, 0:1], (_NP, _D))
        v128 = jnp.broadcast_to(uv[:, 1:2], (_NP, _D))
        t_out, cnt2 = _sc2(u128, v128, srcp, dstp, zeros_hbm)
        ts.append(t_out[:, :eperw, 0].reshape(E))
        reach = jnp.where(cnt2[0, :, 0] + cnt2[1, :, 0] > 0.0, 1.0, 0.0)
        s2 = reach.reshape(_NP, 1)

    alpha = jax.nn.sigmoid(ts[-1])
    seg_max = jax.ops.segment_max(alpha, dst, num_segments=_NP)
    m128 = jnp.broadcast_to(seg_max.reshape(_NP, 1), (_NP, _D))
    m_e = _sc3(m128, dstp)[:, :eperw, 0].reshape(E)
    match = alpha == m_e
    eidx = jnp.where(match, jnp.arange(E, dtype=jnp.int32), jnp.int32(E + _NP))
    first = jax.ops.segment_min(eidx, dst, num_segments=_NP)
    own = jnp.arange(_NP, dtype=jnp.int32)
    ext_idx = jnp.where(first < E, first, E + own)
    parents = jnp.take(srcx, ext_idx)[:N]

    Eb = E // D
    t0r = ts[0].reshape(Eb, D)
    t1r = ts[1].reshape(Eb, D)
    ef = edges.astype(jnp.float32).reshape(Eb, D)
    h0 = edges_h[1].astype(jnp.float32).reshape(Eb, D)
    h1 = edges_h[2].astype(jnp.float32).reshape(Eb, D)

    def padf(x, val):
        return jnp.concatenate([x, jnp.full((npad,), val, x.dtype)]).reshape(_NP // D, D)

    reach_p = reach.reshape(_NP // D, D)
    rt_p = padf(reach_h[-1].astype(jnp.float32), 0.5)
    nmask = padf(jnp.ones((N,), jnp.float32), 0.0)
    par_p = padf(parents.astype(jnp.int32), -2)
    pi_p = padf(pi.astype(jnp.int32), -1)
    return _tc3(t0r, t1r, ef, h0, h1, reach_p, rt_p, nmask, par_p, pi_p)
